# baseline (speedup 1.0000x reference)
import functools

import jax
import jax.numpy as jnp
from jax import lax
from jax.experimental import pallas as pl
from jax.experimental.pallas import tpu as pltpu

N = 256
DH = 128
DE = 64
TJ = 32
EDGE_THRESHOLD = 0.5


def _relu(x):
    return jnp.maximum(x, 0.0)


def _dot(a, b):
    return jnp.dot(a, b, preferred_element_type=jnp.float32)


def _dot00(a, b):
    return lax.dot_general(a, b, (((0,), (0,)), ((), ())),
                           preferred_element_type=jnp.float32)


def _dot01(a, b):
    return lax.dot_general(a, b, (((0,), (1,)), ((), ())),
                           preferred_element_type=jnp.float32)


def _encoder_body(nf_ref, w_ref, m_ref, wt_ref, mt_ref,
                  w1e_ref, b1e_ref, w2e_ref, b2e_ref,
                  l0_refs, l1_refs,
                  w1r_ref, b1r_ref, w2r_ref, b2r_ref,
                  node_out_ref, graph_out_ref,
                  a_sc, s_sc, wd_sc, wmT_sc, A_sc, Db_sc, Sd_sc, Sb_sc, R_sc,
                  EtA_sc, EtB_sc):
    f32 = jnp.float32
    Wv = w_ref[...]
    Mv = m_ref[...]
    absW = jnp.abs(Wv)
    a_v = absW + 0.5 * jnp.abs(Mv)
    s_v = jnp.sign(Wv)
    a_sc[...] = a_v
    s_sc[...] = s_v
    wd_sc[...] = absW * (absW > EDGE_THRESHOLD).astype(f32)

    rr = lax.broadcasted_iota(jnp.int32, (N, N), 0)
    cc = lax.broadcasted_iota(jnp.int32, (N, N), 1)
    offdiag = (rr != cc).astype(f32)
    absMT = jnp.abs(mt_ref[...])
    wmT_sc[...] = (absMT * (absMT > EDGE_THRESHOLD).astype(f32)
                   * offdiag)

    absWT = jnp.abs(wt_ref[...])
    wdT = absWT * (absWT > EDGE_THRESHOLD).astype(f32)
    rs_d = wdT.sum(axis=1, keepdims=True)
    absM = jnp.abs(Mv)
    wm = absM * (absM > EDGE_THRESHOLD).astype(f32) * offdiag
    rs_b = wm.sum(axis=1, keepdims=True)

    def r_plane(k, carry):
        u_k = w1e_ref[0, k]
        v_k = w1e_ref[1, k]
        b_k = b1e_ref[0, k]
        R_sc[pl.ds(k, 1), :, :] = _relu(a_v * u_k + s_v * v_k
                                        + b_k).astype(jnp.bfloat16).reshape(1, N, N)
        return carry

    lax.fori_loop(0, DE, r_plane, 0)

    w2e = w2e_ref[...]
    b2e = b2e_ref[...]

    h = nf_ref[...]
    for lrefs in (l0_refs, l1_refs):
        (w1n_ref, b1n_ref, w2n_ref, b2n_ref,
         w1d_ref, b1d_ref, w2d_ref, b2d_ref,
         w1b_ref, b1b_ref, w2b_ref, b2b_ref,
         g_ref, bb_ref) = lrefs

        A_sc[...] = _dot(h, w1d_ref[0:DH, :])
        BT = _dot01(w1d_ref[DH:2 * DH, :], h)
        We = w1d_ref[2 * DH:2 * DH + DE, :]
        Ge = _dot(w2e, We).astype(jnp.bfloat16)
        ceT = (_dot01(We, b2e)
               + b1d_ref[...]).reshape(DH, 1, 1)
        Db_sc[...] = _dot(h, w1b_ref[0:DH, :])
        CbT = _dot01(w1b_ref[DH:2 * DH, :], h)
        cbT = b1b_ref[...].reshape(DH, 1, 1)

        Sd_sc[...] = jnp.zeros((DH, N), f32)
        Sb_sc[...] = jnp.zeros((DH, N), f32)

        def edge_mm(t, Ge=Ge):
            j0 = t * TJ
            R_t = R_sc[:, pl.ds(j0, TJ), :]
            return _dot00(Ge, R_t.reshape(DE, TJ * N)).reshape(DH, TJ, N)

        def consume(t, Et, BT=BT, ceT=ceT, CbT=CbT, cbT=cbT):
            j0 = t * TJ
            A_t = A_sc[pl.ds(j0, TJ), :]
            AT_t = A_t.T.reshape(DH, TJ, 1)
            Pd = _relu(Et + AT_t + BT.reshape(DH, 1, N) + ceT)
            wd_t = wd_sc[pl.ds(j0, TJ), :]
            Sd_sc[...] += (wd_t[None, :, :] * Pd).sum(axis=1)

            Db_t = Db_sc[pl.ds(j0, TJ), :]
            DbT_t = Db_t.T.reshape(DH, TJ, 1)
            Pb = _relu(DbT_t + CbT.reshape(DH, 1, N) + cbT)
            wmT_t = wmT_sc[pl.ds(j0, TJ), :]
            Sb_sc[...] += (wmT_t[None, :, :] * Pb).sum(axis=1)

        NT = N // TJ
        EtA_sc[...] = edge_mm(0)

        def pair_step(u, carry):
            t0 = 2 * u
            EtB_sc[...] = edge_mm(t0 + 1)
            consume(t0, EtA_sc[...])

            @pl.when(t0 + 2 < NT)
            def _():
                EtA_sc[...] = edge_mm(t0 + 2)

            consume(t0 + 1, EtB_sc[...])
            return carry

        lax.fori_loop(0, NT // 2, pair_step, 0)

        h_dir = _dot00(Sd_sc[...], w2d_ref[...]) + rs_d * b2d_ref[...]
        h_bid = _dot00(Sb_sc[...], w2b_ref[...]) + rs_b * b2b_ref[...]
        h_self = _dot(_relu(_dot(h, w1n_ref[...]) + b1n_ref[...]),
                      w2n_ref[...]) + b2n_ref[...]

        x = h_self + h_dir + h_bid
        mu = x.mean(axis=-1, keepdims=True)
        xc = x - mu
        var = (xc * xc).mean(axis=-1, keepdims=True)
        h = _relu(xc / jnp.sqrt(var + 1e-5) * g_ref[...] + bb_ref[...])

    node_out_ref[...] = h
    gm = h.mean(axis=0, keepdims=True)
    graph_out_ref[...] = (_dot(_relu(_dot(gm, w1r_ref[...]) + b1r_ref[...]),
                               w2r_ref[...]) + b2r_ref[...])


def _body(*refs):
    nf, w, m, wt, mt, w1e, b1e, w2e, b2e = refs[:9]
    l0 = refs[9:23]
    l1 = refs[23:37]
    w1r, b1r, w2r, b2r = refs[37:41]
    node_out, graph_out = refs[41:43]
    scratch = refs[43:]
    _encoder_body(nf, w, m, wt, mt, w1e, b1e, w2e, b2e, l0, l1,
                  w1r, b1r, w2r, b2r, node_out, graph_out, *scratch)


@functools.partial(jax.jit, static_argnames=("interpret",))
def _run(operands, interpret=False):
    out_shapes = (jax.ShapeDtypeStruct((N, DH), jnp.float32),
                  jax.ShapeDtypeStruct((1, DH), jnp.float32))
    scratch = ([pltpu.VMEM((N, N), jnp.float32)] * 4
               + [pltpu.VMEM((N, DH), jnp.float32)] * 2
               + [pltpu.VMEM((DH, N), jnp.float32)] * 2
               + [pltpu.VMEM((DE, N, N), jnp.bfloat16)]
               + [pltpu.VMEM((DH, TJ, N), jnp.float32)] * 2)
    n_in = 41
    in_specs = [pl.BlockSpec(memory_space=pltpu.VMEM) for _ in range(n_in)]
    in_specs[5] = pl.BlockSpec(memory_space=pltpu.SMEM)
    in_specs[6] = pl.BlockSpec(memory_space=pltpu.SMEM)
    return pl.pallas_call(_body, out_shape=out_shapes,
                          in_specs=in_specs,
                          scratch_shapes=scratch,
                          interpret=interpret)(*operands)


def _row(x):
    return x.reshape(1, -1)


def _col(x):
    return x.reshape(-1, 1)


def kernel(node_features, W, M, edge_enc, layers, readout, interpret=False):
    (w1e, b1e), (w2e, b2e) = edge_enc
    (w1r, b1r), (w2r, b2r) = readout
    operands = [node_features, W, M, W.T, M.T,
                w1e, _row(b1e), w2e, _row(b2e)]
    for (node_p, d_p, b_p, (g, bb)) in layers:
        (w1n, b1n), (w2n, b2n) = node_p
        (w1d, b1d), (w2d, b2d) = d_p
        (w1b, b1b), (w2b, b2b) = b_p
        operands += [w1n, _row(b1n), w2n, _row(b2n),
                     w1d, _col(b1d), w2d, _row(b2d),
                     w1b, _col(b1b), w2b, _row(b2b),
                     _row(g), _row(bb)]
    operands += [w1r, _row(b1r), w2r, _row(b2r)]
    node_emb, graph_emb = _run(tuple(operands), interpret=interpret)
    return node_emb, graph_emb.reshape(DH)


# speedup vs baseline: 1.0745x; 1.0745x over previous
import functools

import jax
import jax.numpy as jnp
from jax import lax
from jax.experimental import pallas as pl
from jax.experimental.pallas import tpu as pltpu

N = 256
DH = 128
DE = 64
TJ = 64
EDGE_THRESHOLD = 0.5


def _relu(x):
    return jnp.maximum(x, 0.0)


def _dot(a, b):
    return jnp.dot(a, b, preferred_element_type=jnp.float32)


def _dot00(a, b):
    return lax.dot_general(a, b, (((0,), (0,)), ((), ())),
                           preferred_element_type=jnp.float32)


def _dot01(a, b):
    return lax.dot_general(a, b, (((0,), (1,)), ((), ())),
                           preferred_element_type=jnp.float32)


def _encoder_body(nf_ref, w_ref, m_ref, wt_ref, mt_ref,
                  w1e_ref, b1e_ref, w2e_ref, b2e_ref,
                  l0_refs, l1_refs,
                  w1r_ref, b1r_ref, w2r_ref, b2r_ref,
                  node_out_ref, graph_out_ref,
                  a_sc, s_sc, wd_sc, wmT_sc, A_sc, Db_sc, Sd_sc, Sb_sc, R_sc,
                  EtA_sc, EtB_sc):
    f32 = jnp.float32
    Wv = w_ref[...]
    Mv = m_ref[...]
    absW = jnp.abs(Wv)
    a_v = absW + 0.5 * jnp.abs(Mv)
    s_v = jnp.sign(Wv)
    a_sc[...] = a_v
    s_sc[...] = s_v
    wd_sc[...] = absW * (absW > EDGE_THRESHOLD).astype(f32)

    rr = lax.broadcasted_iota(jnp.int32, (N, N), 0)
    cc = lax.broadcasted_iota(jnp.int32, (N, N), 1)
    offdiag = (rr != cc).astype(f32)
    absMT = jnp.abs(mt_ref[...])
    wmT_sc[...] = (absMT * (absMT > EDGE_THRESHOLD).astype(f32)
                   * offdiag)

    absWT = jnp.abs(wt_ref[...])
    wdT = absWT * (absWT > EDGE_THRESHOLD).astype(f32)
    rs_d = wdT.sum(axis=1, keepdims=True)
    absM = jnp.abs(Mv)
    wm = absM * (absM > EDGE_THRESHOLD).astype(f32) * offdiag
    rs_b = wm.sum(axis=1, keepdims=True)

    def r_plane(k, carry):
        u_k = w1e_ref[0, k]
        v_k = w1e_ref[1, k]
        b_k = b1e_ref[0, k]
        R_sc[pl.ds(k, 1), :, :] = _relu(a_v * u_k + s_v * v_k
                                        + b_k).astype(jnp.bfloat16).reshape(1, N, N)
        return carry

    lax.fori_loop(0, DE, r_plane, 0)

    w2e = w2e_ref[...]
    b2e = b2e_ref[...]

    h = nf_ref[...]
    for lrefs in (l0_refs, l1_refs):
        (w1n_ref, b1n_ref, w2n_ref, b2n_ref,
         w1d_ref, b1d_ref, w2d_ref, b2d_ref,
         w1b_ref, b1b_ref, w2b_ref, b2b_ref,
         g_ref, bb_ref) = lrefs

        A_sc[...] = _dot(h, w1d_ref[0:DH, :])
        BT = _dot01(w1d_ref[DH:2 * DH, :], h)
        We = w1d_ref[2 * DH:2 * DH + DE, :]
        Ge = _dot(w2e, We).astype(jnp.bfloat16)
        ceT = (_dot01(We, b2e)
               + b1d_ref[...]).reshape(DH, 1, 1)
        Db_sc[...] = _dot(h, w1b_ref[0:DH, :])
        CbT = _dot01(w1b_ref[DH:2 * DH, :], h)
        cbT = b1b_ref[...].reshape(DH, 1, 1)

        Sd_sc[...] = jnp.zeros((DH, N), f32)
        Sb_sc[...] = jnp.zeros((DH, N), f32)

        def edge_mm(t, Ge=Ge):
            j0 = t * TJ
            R_t = R_sc[:, pl.ds(j0, TJ), :]
            return _dot00(Ge, R_t.reshape(DE, TJ * N)).reshape(DH, TJ, N)

        def consume(t, Et, BT=BT, ceT=ceT, CbT=CbT, cbT=cbT):
            j0 = t * TJ
            A_t = A_sc[pl.ds(j0, TJ), :]
            AT_t = A_t.T.reshape(DH, TJ, 1)
            Pd = _relu(Et + AT_t + BT.reshape(DH, 1, N) + ceT)
            wd_t = wd_sc[pl.ds(j0, TJ), :]
            Sd_sc[...] += (wd_t[None, :, :] * Pd).sum(axis=1)

            Db_t = Db_sc[pl.ds(j0, TJ), :]
            DbT_t = Db_t.T.reshape(DH, TJ, 1)
            Pb = _relu(DbT_t + CbT.reshape(DH, 1, N) + cbT)
            wmT_t = wmT_sc[pl.ds(j0, TJ), :]
            Sb_sc[...] += (wmT_t[None, :, :] * Pb).sum(axis=1)

        NT = N // TJ
        EtA_sc[...] = edge_mm(0)

        def pair_step(u, carry):
            t0 = 2 * u
            EtB_sc[...] = edge_mm(t0 + 1)
            consume(t0, EtA_sc[...])

            @pl.when(t0 + 2 < NT)
            def _():
                EtA_sc[...] = edge_mm(t0 + 2)

            consume(t0 + 1, EtB_sc[...])
            return carry

        lax.fori_loop(0, NT // 2, pair_step, 0)

        h_dir = _dot00(Sd_sc[...], w2d_ref[...]) + rs_d * b2d_ref[...]
        h_bid = _dot00(Sb_sc[...], w2b_ref[...]) + rs_b * b2b_ref[...]
        h_self = _dot(_relu(_dot(h, w1n_ref[...]) + b1n_ref[...]),
                      w2n_ref[...]) + b2n_ref[...]

        x = h_self + h_dir + h_bid
        mu = x.mean(axis=-1, keepdims=True)
        xc = x - mu
        var = (xc * xc).mean(axis=-1, keepdims=True)
        h = _relu(xc / jnp.sqrt(var + 1e-5) * g_ref[...] + bb_ref[...])

    node_out_ref[...] = h
    gm = h.mean(axis=0, keepdims=True)
    graph_out_ref[...] = (_dot(_relu(_dot(gm, w1r_ref[...]) + b1r_ref[...]),
                               w2r_ref[...]) + b2r_ref[...])


def _body(*refs):
    nf, w, m, wt, mt, w1e, b1e, w2e, b2e = refs[:9]
    l0 = refs[9:23]
    l1 = refs[23:37]
    w1r, b1r, w2r, b2r = refs[37:41]
    node_out, graph_out = refs[41:43]
    scratch = refs[43:]
    _encoder_body(nf, w, m, wt, mt, w1e, b1e, w2e, b2e, l0, l1,
                  w1r, b1r, w2r, b2r, node_out, graph_out, *scratch)


@functools.partial(jax.jit, static_argnames=("interpret",))
def _run(operands, interpret=False):
    out_shapes = (jax.ShapeDtypeStruct((N, DH), jnp.float32),
                  jax.ShapeDtypeStruct((1, DH), jnp.float32))
    scratch = ([pltpu.VMEM((N, N), jnp.float32)] * 4
               + [pltpu.VMEM((N, DH), jnp.float32)] * 2
               + [pltpu.VMEM((DH, N), jnp.float32)] * 2
               + [pltpu.VMEM((DE, N, N), jnp.bfloat16)]
               + [pltpu.VMEM((DH, TJ, N), jnp.float32)] * 2)
    n_in = 41
    in_specs = [pl.BlockSpec(memory_space=pltpu.VMEM) for _ in range(n_in)]
    in_specs[5] = pl.BlockSpec(memory_space=pltpu.SMEM)
    in_specs[6] = pl.BlockSpec(memory_space=pltpu.SMEM)
    return pl.pallas_call(_body, out_shape=out_shapes,
                          in_specs=in_specs,
                          scratch_shapes=scratch,
                          interpret=interpret)(*operands)


def _row(x):
    return x.reshape(1, -1)


def _col(x):
    return x.reshape(-1, 1)


def kernel(node_features, W, M, edge_enc, layers, readout, interpret=False):
    (w1e, b1e), (w2e, b2e) = edge_enc
    (w1r, b1r), (w2r, b2r) = readout
    operands = [node_features, W, M, W.T, M.T,
                w1e, _row(b1e), w2e, _row(b2e)]
    for (node_p, d_p, b_p, (g, bb)) in layers:
        (w1n, b1n), (w2n, b2n) = node_p
        (w1d, b1d), (w2d, b2d) = d_p
        (w1b, b1b), (w2b, b2b) = b_p
        operands += [w1n, _row(b1n), w2n, _row(b2n),
                     w1d, _col(b1d), w2d, _row(b2d),
                     w1b, _col(b1b), w2b, _row(b2b),
                     _row(g), _row(bb)]
    operands += [w1r, _row(b1r), w2r, _row(b2r)]
    node_emb, graph_emb = _run(tuple(operands), interpret=interpret)
    return node_emb, graph_emb.reshape(DH)


# speedup vs baseline: 1.1477x; 1.0681x over previous
import functools

import jax
import jax.numpy as jnp
from jax import lax
from jax.experimental import pallas as pl
from jax.experimental.pallas import tpu as pltpu

N = 256
DH = 128
DE = 64
TJ = 64
EDGE_THRESHOLD = 0.5


def _relu(x):
    return jnp.maximum(x, 0.0)


def _dot(a, b):
    return jnp.dot(a, b, preferred_element_type=jnp.float32)


def _dot00(a, b):
    return lax.dot_general(a, b, (((0,), (0,)), ((), ())),
                           preferred_element_type=jnp.float32)


def _dot01(a, b):
    return lax.dot_general(a, b, (((0,), (1,)), ((), ())),
                           preferred_element_type=jnp.float32)


def _encoder_body(nf_ref, w_ref, m_ref, wt_ref, mt_ref,
                  w1e_ref, b1e_ref, w2e_ref, b2e_ref,
                  l0_refs, l1_refs,
                  w1r_ref, b1r_ref, w2r_ref, b2r_ref,
                  node_out_ref, graph_out_ref,
                  a_sc, s_sc, wd_sc, wmT_sc, A_sc, Db_sc, Sd_sc, Sb_sc, R_sc,
                  EtA_sc, EtB_sc):
    f32 = jnp.float32
    Wv = w_ref[...]
    Mv = m_ref[...]
    absW = jnp.abs(Wv)
    a_v = absW + 0.5 * jnp.abs(Mv)
    s_v = jnp.sign(Wv)
    a_sc[...] = a_v
    s_sc[...] = s_v
    wd_sc[...] = absW * (absW > EDGE_THRESHOLD).astype(f32)

    rr = lax.broadcasted_iota(jnp.int32, (N, N), 0)
    cc = lax.broadcasted_iota(jnp.int32, (N, N), 1)
    offdiag = (rr != cc).astype(f32)
    absMT = jnp.abs(mt_ref[...])
    wmT_sc[...] = (absMT * (absMT > EDGE_THRESHOLD).astype(f32)
                   * offdiag)

    absWT = jnp.abs(wt_ref[...])
    wdT = absWT * (absWT > EDGE_THRESHOLD).astype(f32)
    rs_d = wdT.sum(axis=1, keepdims=True)
    absM = jnp.abs(Mv)
    wm = absM * (absM > EDGE_THRESHOLD).astype(f32) * offdiag
    rs_b = wm.sum(axis=1, keepdims=True)

    def r_plane(k, carry):
        u_k = w1e_ref[0, k]
        v_k = w1e_ref[1, k]
        b_k = b1e_ref[0, k]
        R_sc[pl.ds(k, 1), :, :] = _relu(a_v * u_k + s_v * v_k
                                        + b_k).astype(jnp.bfloat16).reshape(1, N, N)
        return carry

    lax.fori_loop(0, DE, r_plane, 0)

    w2e = w2e_ref[...]
    b2e = b2e_ref[...]

    h = nf_ref[...]
    for lrefs in (l0_refs, l1_refs):
        (w1n_ref, b1n_ref, w2n_ref, b2n_ref,
         w1d_ref, b1d_ref, w2d_ref, b2d_ref,
         w1b_ref, b1b_ref, w2b_ref, b2b_ref,
         g_ref, bb_ref) = lrefs

        A_sc[...] = _dot(h, w1d_ref[0:DH, :])
        BT = _dot01(w1d_ref[DH:2 * DH, :], h)
        We = w1d_ref[2 * DH:2 * DH + DE, :]
        Ge = _dot(w2e, We).astype(jnp.bfloat16)
        BT = BT + _dot01(We, b2e) + b1d_ref[...]
        Db_sc[...] = _dot(h, w1b_ref[0:DH, :])
        CbT = _dot01(w1b_ref[DH:2 * DH, :], h) + b1b_ref[...]

        Sd_sc[...] = jnp.zeros((DH, N), f32)
        Sb_sc[...] = jnp.zeros((DH, N), f32)

        def edge_mm(t, Ge=Ge):
            j0 = t * TJ
            R_t = R_sc[:, pl.ds(j0, TJ), :]
            return _dot00(Ge, R_t.reshape(DE, TJ * N)).reshape(DH, TJ, N)

        def consume(t, Et, BT=BT, CbT=CbT):
            j0 = t * TJ
            A_t = A_sc[pl.ds(j0, TJ), :]
            AT_t = A_t.T.reshape(DH, TJ, 1)
            Pd = _relu(Et + AT_t + BT.reshape(DH, 1, N))
            wd_t = wd_sc[pl.ds(j0, TJ), :]
            Sd_sc[...] += (wd_t[None, :, :] * Pd).sum(axis=1)

            Db_t = Db_sc[pl.ds(j0, TJ), :]
            DbT_t = Db_t.T.reshape(DH, TJ, 1)
            Pb = _relu(DbT_t + CbT.reshape(DH, 1, N))
            wmT_t = wmT_sc[pl.ds(j0, TJ), :]
            Sb_sc[...] += (wmT_t[None, :, :] * Pb).sum(axis=1)

        NT = N // TJ
        EtA_sc[...] = edge_mm(0)

        def pair_step(u, carry):
            t0 = 2 * u
            EtB_sc[...] = edge_mm(t0 + 1)
            consume(t0, EtA_sc[...])

            @pl.when(t0 + 2 < NT)
            def _():
                EtA_sc[...] = edge_mm(t0 + 2)

            consume(t0 + 1, EtB_sc[...])
            return carry

        lax.fori_loop(0, NT // 2, pair_step, 0)

        h_dir = _dot00(Sd_sc[...], w2d_ref[...]) + rs_d * b2d_ref[...]
        h_bid = _dot00(Sb_sc[...], w2b_ref[...]) + rs_b * b2b_ref[...]
        h_self = _dot(_relu(_dot(h, w1n_ref[...]) + b1n_ref[...]),
                      w2n_ref[...]) + b2n_ref[...]

        x = h_self + h_dir + h_bid
        mu = x.mean(axis=-1, keepdims=True)
        xc = x - mu
        var = (xc * xc).mean(axis=-1, keepdims=True)
        h = _relu(xc / jnp.sqrt(var + 1e-5) * g_ref[...] + bb_ref[...])

    node_out_ref[...] = h
    gm = h.mean(axis=0, keepdims=True)
    graph_out_ref[...] = (_dot(_relu(_dot(gm, w1r_ref[...]) + b1r_ref[...]),
                               w2r_ref[...]) + b2r_ref[...])


def _body(*refs):
    nf, w, m, wt, mt, w1e, b1e, w2e, b2e = refs[:9]
    l0 = refs[9:23]
    l1 = refs[23:37]
    w1r, b1r, w2r, b2r = refs[37:41]
    node_out, graph_out = refs[41:43]
    scratch = refs[43:]
    _encoder_body(nf, w, m, wt, mt, w1e, b1e, w2e, b2e, l0, l1,
                  w1r, b1r, w2r, b2r, node_out, graph_out, *scratch)


@functools.partial(jax.jit, static_argnames=("interpret",))
def _run(operands, interpret=False):
    out_shapes = (jax.ShapeDtypeStruct((N, DH), jnp.float32),
                  jax.ShapeDtypeStruct((1, DH), jnp.float32))
    scratch = ([pltpu.VMEM((N, N), jnp.float32)] * 4
               + [pltpu.VMEM((N, DH), jnp.float32)] * 2
               + [pltpu.VMEM((DH, N), jnp.float32)] * 2
               + [pltpu.VMEM((DE, N, N), jnp.bfloat16)]
               + [pltpu.VMEM((DH, TJ, N), jnp.float32)] * 2)
    n_in = 41
    in_specs = [pl.BlockSpec(memory_space=pltpu.VMEM) for _ in range(n_in)]
    in_specs[5] = pl.BlockSpec(memory_space=pltpu.SMEM)
    in_specs[6] = pl.BlockSpec(memory_space=pltpu.SMEM)
    return pl.pallas_call(_body, out_shape=out_shapes,
                          in_specs=in_specs,
                          scratch_shapes=scratch,
                          interpret=interpret)(*operands)


def _row(x):
    return x.reshape(1, -1)


def _col(x):
    return x.reshape(-1, 1)


def kernel(node_features, W, M, edge_enc, layers, readout, interpret=False):
    (w1e, b1e), (w2e, b2e) = edge_enc
    (w1r, b1r), (w2r, b2r) = readout
    operands = [node_features, W, M, W.T, M.T,
                w1e, _row(b1e), w2e, _row(b2e)]
    for (node_p, d_p, b_p, (g, bb)) in layers:
        (w1n, b1n), (w2n, b2n) = node_p
        (w1d, b1d), (w2d, b2d) = d_p
        (w1b, b1b), (w2b, b2b) = b_p
        operands += [w1n, _row(b1n), w2n, _row(b2n),
                     w1d, _col(b1d), w2d, _row(b2d),
                     w1b, _col(b1b), w2b, _row(b2b),
                     _row(g), _row(bb)]
    operands += [w1r, _row(b1r), w2r, _row(b2r)]
    node_emb, graph_emb = _run(tuple(operands), interpret=interpret)
    return node_emb, graph_emb.reshape(DH)


# speedup vs baseline: 1.2263x; 1.0685x over previous
import functools

import jax
import jax.numpy as jnp
from jax import lax
from jax.experimental import pallas as pl
from jax.experimental.pallas import tpu as pltpu

N = 256
DH = 128
DE = 64
TJ = 64
EDGE_THRESHOLD = 0.5


def _relu(x):
    return jnp.maximum(x, 0.0)


def _dot(a, b):
    return jnp.dot(a, b, preferred_element_type=jnp.float32)


def _dot00(a, b):
    return lax.dot_general(a, b, (((0,), (0,)), ((), ())),
                           preferred_element_type=jnp.float32)


def _dot01(a, b):
    return lax.dot_general(a, b, (((0,), (1,)), ((), ())),
                           preferred_element_type=jnp.float32)


def _encoder_body(nf_ref, w_ref, m_ref, wt_ref, mt_ref,
                  w1e_ref, b1e_ref, w2e_ref, b2e_ref,
                  l0_refs, l1_refs,
                  w1r_ref, b1r_ref, w2r_ref, b2r_ref,
                  node_out_ref, graph_out_ref,
                  a_sc, s_sc, wd_sc, wmT_sc, A_sc, Db_sc, Sd_sc, Sb_sc, R_sc,
                  EtA_sc, EtB_sc):
    f32 = jnp.float32
    Wv = w_ref[...]
    Mv = m_ref[...]
    absW = jnp.abs(Wv)
    a_v = absW + 0.5 * jnp.abs(Mv)
    s_v = jnp.sign(Wv)
    a_sc[...] = a_v
    s_sc[...] = s_v
    wd_sc[...] = (absW * (absW > EDGE_THRESHOLD).astype(f32)
                  ).astype(jnp.bfloat16)

    rr = lax.broadcasted_iota(jnp.int32, (N, N), 0)
    cc = lax.broadcasted_iota(jnp.int32, (N, N), 1)
    offdiag = (rr != cc).astype(f32)
    absMT = jnp.abs(mt_ref[...])
    wmT_sc[...] = (absMT * (absMT > EDGE_THRESHOLD).astype(f32)
                   * offdiag).astype(jnp.bfloat16)

    absWT = jnp.abs(wt_ref[...])
    wdT = absWT * (absWT > EDGE_THRESHOLD).astype(f32)
    rs_d = wdT.sum(axis=1, keepdims=True)
    absM = jnp.abs(Mv)
    wm = absM * (absM > EDGE_THRESHOLD).astype(f32) * offdiag
    rs_b = wm.sum(axis=1, keepdims=True)

    def r_plane(k, carry):
        u_k = w1e_ref[0, k]
        v_k = w1e_ref[1, k]
        b_k = b1e_ref[0, k]
        R_sc[pl.ds(k, 1), :, :] = _relu(a_v * u_k + s_v * v_k
                                        + b_k).astype(jnp.bfloat16).reshape(1, N, N)
        return carry

    lax.fori_loop(0, DE, r_plane, 0)

    w2e = w2e_ref[...]
    b2e = b2e_ref[...]

    h = nf_ref[...]
    for lrefs in (l0_refs, l1_refs):
        (w1n_ref, b1n_ref, w2n_ref, b2n_ref,
         w1d_ref, b1d_ref, w2d_ref, b2d_ref,
         w1b_ref, b1b_ref, w2b_ref, b2b_ref,
         g_ref, bb_ref) = lrefs

        A_sc[...] = _dot(h, w1d_ref[0:DH, :])
        BT = _dot01(w1d_ref[DH:2 * DH, :], h)
        We = w1d_ref[2 * DH:2 * DH + DE, :]
        Ge = _dot(w2e, We).astype(jnp.bfloat16)
        BT = BT + _dot01(We, b2e) + b1d_ref[...]
        Db_sc[...] = _dot(h, w1b_ref[0:DH, :])
        CbT = _dot01(w1b_ref[DH:2 * DH, :], h) + b1b_ref[...]

        Sd_sc[...] = jnp.zeros((DH, N), f32)
        Sb_sc[...] = jnp.zeros((DH, N), f32)

        def edge_mm(t, Ge=Ge):
            j0 = t * TJ
            R_t = R_sc[:, pl.ds(j0, TJ), :]
            return _dot00(Ge, R_t.reshape(DE, TJ * N)).astype(
                jnp.bfloat16).reshape(DH, TJ, N)

        def consume(t, Et, BT=BT.astype(jnp.bfloat16), CbT=CbT.astype(jnp.bfloat16)):
            j0 = t * TJ
            A_t = A_sc[pl.ds(j0, TJ), :]
            AT_t = A_t.T.astype(jnp.bfloat16).reshape(DH, TJ, 1)
            Pd = _relu(Et + AT_t + BT.reshape(DH, 1, N))
            wd_t = wd_sc[pl.ds(j0, TJ), :]
            Sd_sc[...] += (wd_t[None, :, :] * Pd).sum(axis=1, dtype=jnp.float32)

            Db_t = Db_sc[pl.ds(j0, TJ), :]
            DbT_t = Db_t.T.astype(jnp.bfloat16).reshape(DH, TJ, 1)
            Pb = _relu(DbT_t + CbT.reshape(DH, 1, N))
            wmT_t = wmT_sc[pl.ds(j0, TJ), :]
            Sb_sc[...] += (wmT_t[None, :, :] * Pb).sum(axis=1, dtype=jnp.float32)

        NT = N // TJ
        EtA_sc[...] = edge_mm(0)

        def pair_step(u, carry):
            t0 = 2 * u
            EtB_sc[...] = edge_mm(t0 + 1)
            consume(t0, EtA_sc[...])

            @pl.when(t0 + 2 < NT)
            def _():
                EtA_sc[...] = edge_mm(t0 + 2)

            consume(t0 + 1, EtB_sc[...])
            return carry

        lax.fori_loop(0, NT // 2, pair_step, 0)

        h_dir = _dot00(Sd_sc[...], w2d_ref[...]) + rs_d * b2d_ref[...]
        h_bid = _dot00(Sb_sc[...], w2b_ref[...]) + rs_b * b2b_ref[...]
        h_self = _dot(_relu(_dot(h, w1n_ref[...]) + b1n_ref[...]),
                      w2n_ref[...]) + b2n_ref[...]

        x = h_self + h_dir + h_bid
        mu = x.mean(axis=-1, keepdims=True)
        xc = x - mu
        var = (xc * xc).mean(axis=-1, keepdims=True)
        h = _relu(xc / jnp.sqrt(var + 1e-5) * g_ref[...] + bb_ref[...])

    node_out_ref[...] = h
    gm = h.mean(axis=0, keepdims=True)
    graph_out_ref[...] = (_dot(_relu(_dot(gm, w1r_ref[...]) + b1r_ref[...]),
                               w2r_ref[...]) + b2r_ref[...])


def _body(*refs):
    nf, w, m, wt, mt, w1e, b1e, w2e, b2e = refs[:9]
    l0 = refs[9:23]
    l1 = refs[23:37]
    w1r, b1r, w2r, b2r = refs[37:41]
    node_out, graph_out = refs[41:43]
    scratch = refs[43:]
    _encoder_body(nf, w, m, wt, mt, w1e, b1e, w2e, b2e, l0, l1,
                  w1r, b1r, w2r, b2r, node_out, graph_out, *scratch)


@functools.partial(jax.jit, static_argnames=("interpret",))
def _run(operands, interpret=False):
    out_shapes = (jax.ShapeDtypeStruct((N, DH), jnp.float32),
                  jax.ShapeDtypeStruct((1, DH), jnp.float32))
    scratch = ([pltpu.VMEM((N, N), jnp.float32)] * 2
               + [pltpu.VMEM((N, N), jnp.bfloat16)] * 2
               + [pltpu.VMEM((N, DH), jnp.float32)] * 2
               + [pltpu.VMEM((DH, N), jnp.float32)] * 2
               + [pltpu.VMEM((DE, N, N), jnp.bfloat16)]
               + [pltpu.VMEM((DH, TJ, N), jnp.bfloat16)] * 2)
    n_in = 41
    in_specs = [pl.BlockSpec(memory_space=pltpu.VMEM) for _ in range(n_in)]
    in_specs[5] = pl.BlockSpec(memory_space=pltpu.SMEM)
    in_specs[6] = pl.BlockSpec(memory_space=pltpu.SMEM)
    return pl.pallas_call(_body, out_shape=out_shapes,
                          in_specs=in_specs,
                          scratch_shapes=scratch,
                          interpret=interpret)(*operands)


def _row(x):
    return x.reshape(1, -1)


def _col(x):
    return x.reshape(-1, 1)


def kernel(node_features, W, M, edge_enc, layers, readout, interpret=False):
    (w1e, b1e), (w2e, b2e) = edge_enc
    (w1r, b1r), (w2r, b2r) = readout
    operands = [node_features, W, M, W.T, M.T,
                w1e, _row(b1e), w2e, _row(b2e)]
    for (node_p, d_p, b_p, (g, bb)) in layers:
        (w1n, b1n), (w2n, b2n) = node_p
        (w1d, b1d), (w2d, b2d) = d_p
        (w1b, b1b), (w2b, b2b) = b_p
        operands += [w1n, _row(b1n), w2n, _row(b2n),
                     w1d, _col(b1d), w2d, _row(b2d),
                     w1b, _col(b1b), w2b, _row(b2b),
                     _row(g), _row(bb)]
    operands += [w1r, _row(b1r), w2r, _row(b2r)]
    node_emb, graph_emb = _run(tuple(operands), interpret=interpret)
    return node_emb, graph_emb.reshape(DH)


# speedup vs baseline: 1.2577x; 1.0256x over previous
import functools

import jax
import jax.numpy as jnp
from jax import lax
from jax.experimental import pallas as pl
from jax.experimental.pallas import tpu as pltpu

N = 256
DH = 128
DE = 64
TJ = 128
EDGE_THRESHOLD = 0.5


def _relu(x):
    return jnp.maximum(x, 0.0)


def _dot(a, b):
    return jnp.dot(a, b, preferred_element_type=jnp.float32)


def _dot00(a, b):
    return lax.dot_general(a, b, (((0,), (0,)), ((), ())),
                           preferred_element_type=jnp.float32)


def _dot01(a, b):
    return lax.dot_general(a, b, (((0,), (1,)), ((), ())),
                           preferred_element_type=jnp.float32)


def _encoder_body(nf_ref, w_ref, m_ref, wt_ref, mt_ref,
                  w1e_ref, b1e_ref, w2e_ref, b2e_ref,
                  l0_refs, l1_refs,
                  w1r_ref, b1r_ref, w2r_ref, b2r_ref,
                  node_out_ref, graph_out_ref,
                  a_sc, s_sc, wd_sc, wmT_sc, A_sc, Db_sc, Sd_sc, Sb_sc, R_sc,
                  EtA_sc, EtB_sc):
    f32 = jnp.float32
    Wv = w_ref[...]
    Mv = m_ref[...]
    absW = jnp.abs(Wv)
    a_v = absW + 0.5 * jnp.abs(Mv)
    s_v = jnp.sign(Wv)
    a_sc[...] = a_v
    s_sc[...] = s_v
    wd_sc[...] = (absW * (absW > EDGE_THRESHOLD).astype(f32)
                  ).astype(jnp.bfloat16)

    rr = lax.broadcasted_iota(jnp.int32, (N, N), 0)
    cc = lax.broadcasted_iota(jnp.int32, (N, N), 1)
    offdiag = (rr != cc).astype(f32)
    absMT = jnp.abs(mt_ref[...])
    wmT_sc[...] = (absMT * (absMT > EDGE_THRESHOLD).astype(f32)
                   * offdiag).astype(jnp.bfloat16)

    absWT = jnp.abs(wt_ref[...])
    wdT = absWT * (absWT > EDGE_THRESHOLD).astype(f32)
    rs_d = wdT.sum(axis=1, keepdims=True)
    absM = jnp.abs(Mv)
    wm = absM * (absM > EDGE_THRESHOLD).astype(f32) * offdiag
    rs_b = wm.sum(axis=1, keepdims=True)

    def r_plane(k, carry):
        u_k = w1e_ref[0, k]
        v_k = w1e_ref[1, k]
        b_k = b1e_ref[0, k]
        R_sc[pl.ds(k, 1), :, :] = _relu(a_v * u_k + s_v * v_k
                                        + b_k).astype(jnp.bfloat16).reshape(1, N, N)
        return carry

    lax.fori_loop(0, DE, r_plane, 0)

    w2e = w2e_ref[...]
    b2e = b2e_ref[...]

    h = nf_ref[...]
    for lrefs in (l0_refs, l1_refs):
        (w1n_ref, b1n_ref, w2n_ref, b2n_ref,
         w1d_ref, b1d_ref, w2d_ref, b2d_ref,
         w1b_ref, b1b_ref, w2b_ref, b2b_ref,
         g_ref, bb_ref) = lrefs

        A_sc[...] = _dot(h, w1d_ref[0:DH, :])
        BT = _dot01(w1d_ref[DH:2 * DH, :], h)
        We = w1d_ref[2 * DH:2 * DH + DE, :]
        Ge = _dot(w2e, We).astype(jnp.bfloat16)
        BT = BT + _dot01(We, b2e) + b1d_ref[...]
        Db_sc[...] = _dot(h, w1b_ref[0:DH, :])
        CbT = _dot01(w1b_ref[DH:2 * DH, :], h) + b1b_ref[...]

        Sd_sc[...] = jnp.zeros((DH, N), f32)
        Sb_sc[...] = jnp.zeros((DH, N), f32)

        def edge_mm(t, Ge=Ge):
            j0 = t * TJ
            R_t = R_sc[:, pl.ds(j0, TJ), :]
            return _dot00(Ge, R_t.reshape(DE, TJ * N)).astype(
                jnp.bfloat16).reshape(DH, TJ, N)

        def consume(t, Et, BT=BT.astype(jnp.bfloat16), CbT=CbT.astype(jnp.bfloat16)):
            j0 = t * TJ
            A_t = A_sc[pl.ds(j0, TJ), :]
            AT_t = A_t.T.astype(jnp.bfloat16).reshape(DH, TJ, 1)
            Pd = _relu(Et + AT_t + BT.reshape(DH, 1, N))
            wd_t = wd_sc[pl.ds(j0, TJ), :]
            Sd_sc[...] += (wd_t[None, :, :] * Pd).sum(axis=1, dtype=jnp.float32)

            Db_t = Db_sc[pl.ds(j0, TJ), :]
            DbT_t = Db_t.T.astype(jnp.bfloat16).reshape(DH, TJ, 1)
            Pb = _relu(DbT_t + CbT.reshape(DH, 1, N))
            wmT_t = wmT_sc[pl.ds(j0, TJ), :]
            Sb_sc[...] += (wmT_t[None, :, :] * Pb).sum(axis=1, dtype=jnp.float32)

        NT = N // TJ
        EtA_sc[...] = edge_mm(0)

        def pair_step(u, carry):
            t0 = 2 * u
            EtB_sc[...] = edge_mm(t0 + 1)
            consume(t0, EtA_sc[...])

            @pl.when(t0 + 2 < NT)
            def _():
                EtA_sc[...] = edge_mm(t0 + 2)

            consume(t0 + 1, EtB_sc[...])
            return carry

        lax.fori_loop(0, NT // 2, pair_step, 0)

        h_dir = _dot00(Sd_sc[...], w2d_ref[...]) + rs_d * b2d_ref[...]
        h_bid = _dot00(Sb_sc[...], w2b_ref[...]) + rs_b * b2b_ref[...]
        h_self = _dot(_relu(_dot(h, w1n_ref[...]) + b1n_ref[...]),
                      w2n_ref[...]) + b2n_ref[...]

        x = h_self + h_dir + h_bid
        mu = x.mean(axis=-1, keepdims=True)
        xc = x - mu
        var = (xc * xc).mean(axis=-1, keepdims=True)
        h = _relu(xc / jnp.sqrt(var + 1e-5) * g_ref[...] + bb_ref[...])

    node_out_ref[...] = h
    gm = h.mean(axis=0, keepdims=True)
    graph_out_ref[...] = (_dot(_relu(_dot(gm, w1r_ref[...]) + b1r_ref[...]),
                               w2r_ref[...]) + b2r_ref[...])


def _body(*refs):
    nf, w, m, wt, mt, w1e, b1e, w2e, b2e = refs[:9]
    l0 = refs[9:23]
    l1 = refs[23:37]
    w1r, b1r, w2r, b2r = refs[37:41]
    node_out, graph_out = refs[41:43]
    scratch = refs[43:]
    _encoder_body(nf, w, m, wt, mt, w1e, b1e, w2e, b2e, l0, l1,
                  w1r, b1r, w2r, b2r, node_out, graph_out, *scratch)


@functools.partial(jax.jit, static_argnames=("interpret",))
def _run(operands, interpret=False):
    out_shapes = (jax.ShapeDtypeStruct((N, DH), jnp.float32),
                  jax.ShapeDtypeStruct((1, DH), jnp.float32))
    scratch = ([pltpu.VMEM((N, N), jnp.float32)] * 2
               + [pltpu.VMEM((N, N), jnp.bfloat16)] * 2
               + [pltpu.VMEM((N, DH), jnp.float32)] * 2
               + [pltpu.VMEM((DH, N), jnp.float32)] * 2
               + [pltpu.VMEM((DE, N, N), jnp.bfloat16)]
               + [pltpu.VMEM((DH, TJ, N), jnp.bfloat16)] * 2)
    n_in = 41
    in_specs = [pl.BlockSpec(memory_space=pltpu.VMEM) for _ in range(n_in)]
    in_specs[5] = pl.BlockSpec(memory_space=pltpu.SMEM)
    in_specs[6] = pl.BlockSpec(memory_space=pltpu.SMEM)
    return pl.pallas_call(_body, out_shape=out_shapes,
                          in_specs=in_specs,
                          scratch_shapes=scratch,
                          interpret=interpret)(*operands)


def _row(x):
    return x.reshape(1, -1)


def _col(x):
    return x.reshape(-1, 1)


def kernel(node_features, W, M, edge_enc, layers, readout, interpret=False):
    (w1e, b1e), (w2e, b2e) = edge_enc
    (w1r, b1r), (w2r, b2r) = readout
    operands = [node_features, W, M, W.T, M.T,
                w1e, _row(b1e), w2e, _row(b2e)]
    for (node_p, d_p, b_p, (g, bb)) in layers:
        (w1n, b1n), (w2n, b2n) = node_p
        (w1d, b1d), (w2d, b2d) = d_p
        (w1b, b1b), (w2b, b2b) = b_p
        operands += [w1n, _row(b1n), w2n, _row(b2n),
                     w1d, _col(b1d), w2d, _row(b2d),
                     w1b, _col(b1b), w2b, _row(b2b),
                     _row(g), _row(bb)]
    operands += [w1r, _row(b1r), w2r, _row(b2r)]
    node_emb, graph_emb = _run(tuple(operands), interpret=interpret)
    return node_emb, graph_emb.reshape(DH)


# speedup vs baseline: 1.2585x; 1.0006x over previous
import functools

import jax
import jax.numpy as jnp
from jax import lax
from jax.experimental import pallas as pl
from jax.experimental.pallas import tpu as pltpu

N = 256
DH = 128
DE = 64
TJ = 128
EDGE_THRESHOLD = 0.5


def _relu(x):
    return jnp.maximum(x, 0.0)


def _dot(a, b):
    return jnp.dot(a, b, preferred_element_type=jnp.float32)


def _dot00(a, b):
    return lax.dot_general(a, b, (((0,), (0,)), ((), ())),
                           preferred_element_type=jnp.float32)


def _dot01(a, b):
    return lax.dot_general(a, b, (((0,), (1,)), ((), ())),
                           preferred_element_type=jnp.float32)


def _encoder_body(nf_ref, w_ref, m_ref, wt_ref, mt_ref,
                  w1e_ref, b1e_ref, w2e_ref, b2e_ref,
                  l0_refs, l1_refs,
                  w1r_ref, b1r_ref, w2r_ref, b2r_ref,
                  node_out_ref, graph_out_ref,
                  a_sc, s_sc, wd_sc, wmT_sc, A_sc, Db_sc, Sd_sc, Sb_sc, R_sc,
                  EtA_sc, EtB_sc):
    f32 = jnp.float32
    Wv = w_ref[...]
    Mv = m_ref[...]
    absW = jnp.abs(Wv)
    a_v = absW + 0.5 * jnp.abs(Mv)
    s_v = jnp.sign(Wv)
    a_sc[...] = a_v
    s_sc[...] = s_v
    wd_sc[...] = (absW * (absW > EDGE_THRESHOLD).astype(f32)
                  ).astype(jnp.bfloat16)

    rr = lax.broadcasted_iota(jnp.int32, (N, N), 0)
    cc = lax.broadcasted_iota(jnp.int32, (N, N), 1)
    offdiag = (rr != cc).astype(f32)
    absMT = jnp.abs(mt_ref[...])
    wmT_sc[...] = (absMT * (absMT > EDGE_THRESHOLD).astype(f32)
                   * offdiag).astype(jnp.bfloat16)

    absWT = jnp.abs(wt_ref[...])
    wdT = absWT * (absWT > EDGE_THRESHOLD).astype(f32)
    rs_d = wdT.sum(axis=1, keepdims=True)
    absM = jnp.abs(Mv)
    wm = absM * (absM > EDGE_THRESHOLD).astype(f32) * offdiag
    rs_b = wm.sum(axis=1, keepdims=True)

    def r_plane(k, carry):
        u_k = w1e_ref[0, k]
        v_k = w1e_ref[1, k]
        b_k = b1e_ref[0, k]
        R_sc[pl.ds(k, 1), :, :] = _relu(a_v * u_k + s_v * v_k
                                        + b_k).astype(jnp.bfloat16).reshape(1, N, N)
        return carry

    lax.fori_loop(0, DE, r_plane, 0)

    w2e = w2e_ref[...]
    b2e = b2e_ref[...]

    h = nf_ref[...]
    for lrefs in (l0_refs, l1_refs):
        (w1n_ref, b1n_ref, w2n_ref, b2n_ref,
         w1d_ref, b1d_ref, w2d_ref, b2d_ref,
         w1b_ref, b1b_ref, w2b_ref, b2b_ref,
         g_ref, bb_ref) = lrefs

        A_sc[...] = _dot(h, w1d_ref[0:DH, :])
        BT = _dot01(w1d_ref[DH:2 * DH, :], h)
        We = w1d_ref[2 * DH:2 * DH + DE, :]
        Ge = _dot(w2e, We).astype(jnp.bfloat16)
        BT = BT + _dot01(We, b2e) + b1d_ref[...]
        Db_sc[...] = _dot(h, w1b_ref[0:DH, :])
        CbT = _dot01(w1b_ref[DH:2 * DH, :], h) + b1b_ref[...]

        Sd_sc[...] = jnp.zeros((DH, N), f32)
        Sb_sc[...] = jnp.zeros((DH, N), f32)

        def edge_mm(t, Ge=Ge):
            j0 = t * TJ
            R_t = R_sc[:, pl.ds(j0, TJ), :]
            return _dot00(Ge, R_t.reshape(DE, TJ * N)).astype(
                jnp.bfloat16).reshape(DH, TJ, N)

        def consume(t, Et, BT=BT.astype(jnp.bfloat16), CbT=CbT.astype(jnp.bfloat16)):
            j0 = t * TJ
            A_t = A_sc[pl.ds(j0, TJ), :]
            AT_t = A_t.T.astype(jnp.bfloat16).reshape(DH, TJ, 1)
            Pd = _relu(Et + AT_t + BT.reshape(DH, 1, N))
            wd_t = wd_sc[pl.ds(j0, TJ), :]
            Sd_sc[...] += (wd_t[None, :, :] * Pd).sum(axis=1, dtype=jnp.float32)

            Db_t = Db_sc[pl.ds(j0, TJ), :]
            DbT_t = Db_t.T.astype(jnp.bfloat16).reshape(DH, TJ, 1)
            Pb = _relu(DbT_t + CbT.reshape(DH, 1, N))
            wmT_t = wmT_sc[pl.ds(j0, TJ), :]
            Sb_sc[...] += (wmT_t[None, :, :] * Pb).sum(axis=1, dtype=jnp.float32)

        NT = N // TJ
        EtA_sc[...] = edge_mm(0)

        def pair_step(u, carry):
            t0 = 2 * u
            EtB_sc[...] = edge_mm(t0 + 1)
            consume(t0, EtA_sc[...])

            @pl.when(t0 + 2 < NT)
            def _():
                EtA_sc[...] = edge_mm(t0 + 2)

            consume(t0 + 1, EtB_sc[...])
            return carry

        lax.fori_loop(0, NT // 2, pair_step, 0)

        h_dir = _dot00(Sd_sc[...], w2d_ref[...]) + rs_d * b2d_ref[...]
        h_bid = _dot00(Sb_sc[...], w2b_ref[...]) + rs_b * b2b_ref[...]
        h_self = _dot(_relu(_dot(h, w1n_ref[...]) + b1n_ref[...]),
                      w2n_ref[...]) + b2n_ref[...]

        x = h_self + h_dir + h_bid
        mu = x.mean(axis=-1, keepdims=True)
        xc = x - mu
        var = (xc * xc).mean(axis=-1, keepdims=True)
        h = _relu(xc / jnp.sqrt(var + 1e-5) * g_ref[...] + bb_ref[...])

    node_out_ref[...] = h
    gm = h.mean(axis=0, keepdims=True)
    graph_out_ref[...] = (_dot(_relu(_dot(gm, w1r_ref[...]) + b1r_ref[...]),
                               w2r_ref[...]) + b2r_ref[...])


def _body(*refs):
    nf, w, m, wt, mt, w1e, b1e, w2e, b2e = refs[:9]
    l0 = refs[9:23]
    l1 = refs[23:37]
    w1r, b1r, w2r, b2r = refs[37:41]
    node_out, graph_out = refs[41:43]
    scratch = refs[43:]
    _encoder_body(nf, w, m, wt, mt, w1e, b1e, w2e, b2e, l0, l1,
                  w1r, b1r, w2r, b2r, node_out, graph_out, *scratch)


@jax.jit
def _run(operands):
    out_shapes = (jax.ShapeDtypeStruct((N, DH), jnp.float32),
                  jax.ShapeDtypeStruct((1, DH), jnp.float32))
    scratch = ([pltpu.VMEM((N, N), jnp.float32)] * 2
               + [pltpu.VMEM((N, N), jnp.bfloat16)] * 2
               + [pltpu.VMEM((N, DH), jnp.float32)] * 2
               + [pltpu.VMEM((DH, N), jnp.float32)] * 2
               + [pltpu.VMEM((DE, N, N), jnp.bfloat16)]
               + [pltpu.VMEM((DH, TJ, N), jnp.bfloat16)] * 2)
    n_in = 41
    in_specs = [pl.BlockSpec(memory_space=pltpu.VMEM) for _ in range(n_in)]
    in_specs[5] = pl.BlockSpec(memory_space=pltpu.SMEM)
    in_specs[6] = pl.BlockSpec(memory_space=pltpu.SMEM)
    return pl.pallas_call(_body, out_shape=out_shapes,
                          in_specs=in_specs,
                          scratch_shapes=scratch,
                          )(*operands)


def _row(x):
    return x.reshape(1, -1)


def _col(x):
    return x.reshape(-1, 1)


def kernel(node_features, W, M, edge_enc, layers, readout):
    (w1e, b1e), (w2e, b2e) = edge_enc
    (w1r, b1r), (w2r, b2r) = readout
    operands = [node_features, W, M, W.T, M.T,
                w1e, _row(b1e), w2e, _row(b2e)]
    for (node_p, d_p, b_p, (g, bb)) in layers:
        (w1n, b1n), (w2n, b2n) = node_p
        (w1d, b1d), (w2d, b2d) = d_p
        (w1b, b1b), (w2b, b2b) = b_p
        operands += [w1n, _row(b1n), w2n, _row(b2n),
                     w1d, _col(b1d), w2d, _row(b2d),
                     w1b, _col(b1b), w2b, _row(b2b),
                     _row(g), _row(bb)]
    operands += [w1r, _row(b1r), w2r, _row(b2r)]
    node_emb, graph_emb = _run(tuple(operands))
    return node_emb, graph_emb.reshape(DH)
